# baseline (speedup 1.0000x reference)
import numpy as np
import jax
import jax.numpy as jnp
from jax.experimental import pallas as pl
from jax.experimental.pallas import tpu as pltpu

N_FFT = 4096
TOP_K = 32
F_Q_PAD = 544
N_TB = 4
BISECT_ITERS = 30
_CONJ = (False, True, False, True)


def _section_bins(q, f_q_pad):
    h = q // 2
    fmap = np.full((4, f_q_pad), -1)
    for g in range(h + 1):
        fmap[0, g] = g
    for g in range(1, h):
        fmap[1, g] = q - g
    for g in range(h + 1):
        fmap[2, g] = q + g
    for g in range(h):
        fmap[3, g] = 2 * q - g
    return fmap


def _make_tables(n, f_q_pad, cp=128):
    q = n // 4
    nq = q // 2 + 1
    u = np.arange(q)
    m = np.arange(f_q_pad)
    ph = (np.outer(m, u) % q) * (2.0 * np.pi / q)
    cos_q = np.cos(ph)
    sin_q = np.sin(ph)
    cos_q[nq:] = 0.0
    sin_q[nq:] = 0.0

    fmap = _section_bins(q, f_q_pad)
    cw = np.zeros((4, 4, f_q_pad, cp))
    sw = np.zeros((4, 4, f_q_pad, cp))
    for s in range(4):
        valid = fmap[s] >= 0
        fr = np.where(valid, fmap[s], 0)
        for j in range(4):
            a = (fr * j % n) * (2.0 * np.pi / n)
            cw[s, j] = np.where(valid, np.cos(a), 0.0)[:, None]
            sw[s, j] = np.where(valid, np.sin(a), 0.0)[:, None]

    f_pad = 4 * f_q_pad
    t = np.arange(n)
    icos = np.zeros((n, f_pad))
    isin = np.zeros((n, f_pad))
    for s in range(4):
        for g in range(f_q_pad):
            fb = fmap[s, g]
            if fb < 0:
                continue
            r = s * f_q_pad + g
            w = (1.0 / n) if (fb == 0 or fb == n // 2) else (2.0 / n)
            phr = (fb * t % n) * (2.0 * np.pi / n)
            icos[:, r] = w * np.cos(phr)
            isin[:, r] = w * np.sin(phr)

    return (cos_q.astype(np.float32), sin_q.astype(np.float32),
            cw.astype(np.float32), sw.astype(np.float32),
            icos.astype(np.float32), isin.astype(np.float32))


def _make_fwdmask_kernel(top_k, bp):
    def _mask_kernel(cos_ref, sin_ref, cw_ref, sw_ref, x_ref,
                     mre_ref, ms_ref, rps_ref, sps_ref):
        i = pl.program_id(0)

        @pl.when(i < bp)
        def _fwd():
            xc = jnp.concatenate([x_ref[0, 0], x_ref[1, 0]], axis=1)
            q = xc.shape[0] // 4
            xq = xc.reshape(q, 4 * xc.shape[1])
            dn = (((1,), (0,)), ((), ()))
            hp = jax.lax.Precision.HIGHEST
            rps_ref[i % 2] = jax.lax.dot_general(
                cos_ref[...], xq, dn, precision=hp,
                preferred_element_type=jnp.float32)
            sps_ref[i % 2] = jax.lax.dot_general(
                sin_ref[...], xq, dn, precision=hp,
                preferred_element_type=jnp.float32)

        @pl.when(i > 0)
        def _mask():
            rp = rps_ref[(i + 1) % 2]
            sp = sps_ref[(i + 1) % 2]
            cp = rp.shape[-1] // 4
            rea = [rp[:, j * cp:(j + 1) * cp] for j in range(4)]
            sa = [sp[:, j * cp:(j + 1) * cp] for j in range(4)]
            cwt = [[jnp.concatenate([cw_ref[s, j]] * (cp // cw_ref.shape[-1]),
                                    axis=1) for j in range(4)]
                   for s in range(4)]
            swt = [[jnp.concatenate([sw_ref[s, j]] * (cp // sw_ref.shape[-1]),
                                    axis=1) for j in range(4)]
                   for s in range(4)]

            res = []
            for s in range(4):
                re_s = jnp.zeros_like(rea[0])
                s_s = jnp.zeros_like(rea[0])
                for j in range(4):
                    c = cwt[s][j]
                    w = swt[s][j]
                    if _CONJ[s]:
                        re_s = re_s + (c * rea[j] + w * sa[j])
                        s_s = s_s + (w * rea[j] - c * sa[j])
                    else:
                        re_s = re_s + (c * rea[j] - w * sa[j])
                        s_s = s_s + (w * rea[j] + c * sa[j])
                res.append((re_s, s_s, re_s * re_s + s_s * s_s))

            hi = res[0][2].max(axis=0, keepdims=True)
            for s in range(1, 4):
                hi = jnp.maximum(hi, res[s][2].max(axis=0, keepdims=True))
            lo = jnp.full_like(hi, -1.0)

            def body(_, carry):
                lo, hi = carry
                mid = 0.5 * (lo + hi)
                cnt = sum(jnp.sum((mg > mid).astype(jnp.float32), axis=0,
                                  keepdims=True) for _, _, mg in res)
                big = cnt >= top_k
                return jnp.where(big, mid, lo), jnp.where(big, hi, mid)

            lo, hi = jax.lax.fori_loop(0, BISECT_ITERS, body, (lo, hi))
            fq = rea[0].shape[0]
            for s, (re_s, s_s, mg) in enumerate(res):
                keep = mg > lo
                mre_ref[0, s * fq:(s + 1) * fq] = jnp.where(
                    keep, re_s, 0.0).astype(jnp.bfloat16)
                ms_ref[0, s * fq:(s + 1) * fq] = jnp.where(
                    keep, s_s, 0.0).astype(jnp.bfloat16)
    return _mask_kernel


def _inv_kernel(icos_ref, isin_ref, mre_ref, ms_ref, x_ref, xf_ref, res_ref):
    dn = (((1,), (0,)), ((), ()))
    acc = jax.lax.dot_general(icos_ref[...], mre_ref[0], dn,
                              preferred_element_type=jnp.float32)
    acc = acc + jax.lax.dot_general(isin_ref[...], ms_ref[0], dn,
                                    preferred_element_type=jnp.float32)
    c = x_ref.shape[-1]
    xf_ref[0, 0] = acc[:, :c]
    xf_ref[1, 0] = acc[:, c:]
    res_ref[0, 0] = x_ref[0, 0] - acc[:, :c]
    res_ref[1, 0] = x_ref[1, 0] - acc[:, c:]


def _pipeline(x4d, cos_q, sin_q, cw, sw, icos, isin, top_k, interpret=False):
    _, bp, n, c = x4d.shape
    q = n // 4
    cp = 2 * c
    f_q = cos_q.shape[0]
    f_pad = 4 * f_q
    tb = n // N_TB

    bpm = bp - 1
    mre, ms = pl.pallas_call(
        _make_fwdmask_kernel(top_k, bp),
        grid=(bp + 1,),
        in_specs=[
            pl.BlockSpec((f_q, q), lambda j: (0, 0)),
            pl.BlockSpec((f_q, q), lambda j: (0, 0)),
            pl.BlockSpec((4, 4, f_q, cw.shape[-1]),
                         lambda j: (0, 0, 0, 0)),
            pl.BlockSpec((4, 4, f_q, sw.shape[-1]),
                         lambda j: (0, 0, 0, 0)),
            pl.BlockSpec((2, 1, n, c),
                         lambda j: (0, jnp.minimum(j, bpm), 0, 0)),
        ],
        out_specs=[pl.BlockSpec((1, f_pad, cp),
                                lambda j: (jnp.maximum(j - 1, 0), 0, 0))] * 2,
        out_shape=[jax.ShapeDtypeStruct((bp, f_pad, cp), jnp.bfloat16)] * 2,
        scratch_shapes=[pltpu.VMEM((2, f_q, 4 * cp), jnp.float32)] * 2,
        interpret=interpret,
    )(cos_q, sin_q, cw, sw, x4d)

    xf, res = pl.pallas_call(
        _inv_kernel,
        grid=(N_TB, bp),
        in_specs=[
            pl.BlockSpec((tb, f_pad), lambda i, j: (i, 0)),
            pl.BlockSpec((tb, f_pad), lambda i, j: (i, 0)),
            pl.BlockSpec((1, f_pad, cp), lambda i, j: (j, 0, 0)),
            pl.BlockSpec((1, f_pad, cp), lambda i, j: (j, 0, 0)),
            pl.BlockSpec((2, 1, tb, c), lambda i, j: (0, j, i, 0)),
        ],
        out_specs=[pl.BlockSpec((2, 1, tb, c), lambda i, j: (0, j, i, 0))] * 2,
        out_shape=[jax.ShapeDtypeStruct((2, bp, n, c), jnp.float32)] * 2,
        interpret=interpret,
    )(icos.astype(jnp.bfloat16), isin.astype(jnp.bfloat16), mre, ms, x4d)
    return xf, res


_TABLES = _make_tables(N_FFT, F_Q_PAD)


def kernel(x):
    b, n, c = x.shape
    bp = b // 2
    x4d = x.reshape(2, bp, n, c)
    xf, res = _pipeline(x4d, *(jnp.asarray(tbl) for tbl in _TABLES), TOP_K)
    return xf.reshape(b, n, c), res.reshape(b, n, c)


# speedup vs baseline: 1.0990x; 1.0990x over previous
import numpy as np
import jax
import jax.numpy as jnp
from jax.experimental import pallas as pl
from jax.experimental.pallas import tpu as pltpu

N_FFT = 4096
TOP_K = 32
F_Q_PAD = 544
N_TB = 4
BISECT_ITERS = 30
_CONJ = (False, True, False, True)


def _section_bins(q, f_q_pad):
    h = q // 2
    fmap = np.full((4, f_q_pad), -1)
    for g in range(h + 1):
        fmap[0, g] = g
    for g in range(1, h):
        fmap[1, g] = q - g
    for g in range(h + 1):
        fmap[2, g] = q + g
    for g in range(h):
        fmap[3, g] = 2 * q - g
    return fmap


def _make_tables(n, f_q_pad, cp=128):
    q = n // 4
    nq = q // 2 + 1
    u = np.arange(q)
    m = np.arange(f_q_pad)
    ph = (np.outer(m, u) % q) * (2.0 * np.pi / q)
    cos_q = np.cos(ph)
    sin_q = np.sin(ph)
    cos_q[nq:] = 0.0
    sin_q[nq:] = 0.0

    fmap = _section_bins(q, f_q_pad)
    cw = np.zeros((4, 4, f_q_pad, cp))
    sw = np.zeros((4, 4, f_q_pad, cp))
    for s in range(4):
        valid = fmap[s] >= 0
        fr = np.where(valid, fmap[s], 0)
        for j in range(4):
            a = (fr * j % n) * (2.0 * np.pi / n)
            cw[s, j] = np.where(valid, np.cos(a), 0.0)[:, None]
            sw[s, j] = np.where(valid, np.sin(a), 0.0)[:, None]

    f_pad = 4 * f_q_pad
    t = np.arange(n)
    icos = np.zeros((n, f_pad))
    isin = np.zeros((n, f_pad))
    for s in range(4):
        for g in range(f_q_pad):
            fb = fmap[s, g]
            if fb < 0:
                continue
            r = s * f_q_pad + g
            w = (1.0 / n) if (fb == 0 or fb == n // 2) else (2.0 / n)
            phr = (fb * t % n) * (2.0 * np.pi / n)
            icos[:, r] = w * np.cos(phr)
            isin[:, r] = w * np.sin(phr)

    return (cos_q.astype(np.float32), sin_q.astype(np.float32),
            cw.astype(np.float32), sw.astype(np.float32),
            icos.astype(np.float32), isin.astype(np.float32))


def _make_fwdmask_kernel(top_k):
    def _mask_kernel(cos_ref, sin_ref, cw_ref, sw_ref, x_ref,
                     mre_ref, ms_ref):
        xc = jnp.concatenate([x_ref[0, 0], x_ref[1, 0]], axis=1)
        q = xc.shape[0] // 4
        xq = xc.reshape(q, 4 * xc.shape[1])
        dn = (((1,), (0,)), ((), ()))
        hp = jax.lax.Precision.HIGHEST
        rp = jax.lax.dot_general(cos_ref[...], xq, dn, precision=hp,
                                 preferred_element_type=jnp.float32)
        sp = jax.lax.dot_general(sin_ref[...], xq, dn, precision=hp,
                                 preferred_element_type=jnp.float32)
        cp = rp.shape[-1] // 4
        rea = [rp[:, j * cp:(j + 1) * cp] for j in range(4)]
        sa = [sp[:, j * cp:(j + 1) * cp] for j in range(4)]
        cwt = [[jnp.concatenate([cw_ref[s, j]] * (cp // cw_ref.shape[-1]),
                                axis=1) for j in range(4)]
               for s in range(4)]
        swt = [[jnp.concatenate([sw_ref[s, j]] * (cp // sw_ref.shape[-1]),
                                axis=1) for j in range(4)]
               for s in range(4)]

        res = []
        for s in range(4):
            re_s = jnp.zeros_like(rea[0])
            s_s = jnp.zeros_like(rea[0])
            for j in range(4):
                c = cwt[s][j]
                w = swt[s][j]
                if _CONJ[s]:
                    re_s = re_s + (c * rea[j] + w * sa[j])
                    s_s = s_s + (w * rea[j] - c * sa[j])
                else:
                    re_s = re_s + (c * rea[j] - w * sa[j])
                    s_s = s_s + (w * rea[j] + c * sa[j])
            res.append((re_s, s_s, re_s * re_s + s_s * s_s))

        hi = res[0][2].max(axis=0, keepdims=True)
        for s in range(1, 4):
            hi = jnp.maximum(hi, res[s][2].max(axis=0, keepdims=True))
        lo = jnp.full_like(hi, -1.0)

        def body(_, carry):
            lo, hi = carry
            mid = 0.5 * (lo + hi)
            cnt = sum(jnp.sum((mg > mid).astype(jnp.float32), axis=0,
                              keepdims=True) for _, _, mg in res)
            big = cnt >= top_k
            return jnp.where(big, mid, lo), jnp.where(big, hi, mid)

        lo, hi = jax.lax.fori_loop(0, BISECT_ITERS, body, (lo, hi))
        fq = rea[0].shape[0]
        for s, (re_s, s_s, mg) in enumerate(res):
            keep = mg > lo
            mre_ref[0, s * fq:(s + 1) * fq] = jnp.where(
                keep, re_s, 0.0).astype(jnp.bfloat16)
            ms_ref[0, s * fq:(s + 1) * fq] = jnp.where(
                keep, s_s, 0.0).astype(jnp.bfloat16)
    return _mask_kernel


def _inv_kernel(icos_ref, isin_ref, mre_ref, ms_ref, x_ref, xf_ref, res_ref):
    dn = (((1,), (0,)), ((), ()))
    acc = jax.lax.dot_general(icos_ref[...], mre_ref[0], dn,
                              preferred_element_type=jnp.float32)
    acc = acc + jax.lax.dot_general(isin_ref[...], ms_ref[0], dn,
                                    preferred_element_type=jnp.float32)
    c = x_ref.shape[-1]
    xf_ref[0, 0] = acc[:, :c]
    xf_ref[1, 0] = acc[:, c:]
    res_ref[0, 0] = x_ref[0, 0] - acc[:, :c]
    res_ref[1, 0] = x_ref[1, 0] - acc[:, c:]


def _pipeline(x4d, cos_q, sin_q, cw, sw, icos, isin, top_k, interpret=False):
    _, bp, n, c = x4d.shape
    q = n // 4
    cp = 2 * c
    f_q = cos_q.shape[0]
    f_pad = 4 * f_q
    tb = n // N_TB

    mre, ms = pl.pallas_call(
        _make_fwdmask_kernel(top_k),
        grid=(bp,),
        in_specs=[
            pl.BlockSpec((f_q, q), lambda j: (0, 0)),
            pl.BlockSpec((f_q, q), lambda j: (0, 0)),
            pl.BlockSpec((4, 4, f_q, cw.shape[-1]),
                         lambda j: (0, 0, 0, 0)),
            pl.BlockSpec((4, 4, f_q, sw.shape[-1]),
                         lambda j: (0, 0, 0, 0)),
            pl.BlockSpec((2, 1, n, c), lambda j: (0, j, 0, 0)),
        ],
        out_specs=[pl.BlockSpec((1, f_pad, cp), lambda j: (j, 0, 0))] * 2,
        out_shape=[jax.ShapeDtypeStruct((bp, f_pad, cp), jnp.bfloat16)] * 2,
        interpret=interpret,
    )(cos_q, sin_q, cw, sw, x4d)

    xf, res = pl.pallas_call(
        _inv_kernel,
        grid=(N_TB, bp),
        in_specs=[
            pl.BlockSpec((tb, f_pad), lambda i, j: (i, 0)),
            pl.BlockSpec((tb, f_pad), lambda i, j: (i, 0)),
            pl.BlockSpec((1, f_pad, cp), lambda i, j: (j, 0, 0)),
            pl.BlockSpec((1, f_pad, cp), lambda i, j: (j, 0, 0)),
            pl.BlockSpec((2, 1, tb, c), lambda i, j: (0, j, i, 0)),
        ],
        out_specs=[pl.BlockSpec((2, 1, tb, c), lambda i, j: (0, j, i, 0))] * 2,
        out_shape=[jax.ShapeDtypeStruct((2, bp, n, c), jnp.float32)] * 2,
        interpret=interpret,
    )(icos.astype(jnp.bfloat16), isin.astype(jnp.bfloat16), mre, ms, x4d)
    return xf, res


_TABLES = _make_tables(N_FFT, F_Q_PAD)


def kernel(x):
    b, n, c = x.shape
    bp = b // 2
    x4d = x.reshape(2, bp, n, c)
    xf, res = _pipeline(x4d, *(jnp.asarray(tbl) for tbl in _TABLES), TOP_K)
    return xf.reshape(b, n, c), res.reshape(b, n, c)
